# baseline (speedup 1.0000x reference)
import jax
import jax.numpy as jnp
from jax.experimental import pallas as pl
from jax.experimental.pallas import tpu as pltpu

N = 16384
D = 1024
CH = 512
NBUF = 8
NCH = N // CH


def _body(w_ref, b_ref, u_ref, x_hbm, o_hbm, xbuf, obuf, insem, outsem):
    def in_copy(k, b):
        return pltpu.make_async_copy(
            x_hbm.at[pl.ds(k * CH, CH), :], xbuf.at[b], insem.at[b])

    def out_copy(k, b):
        return pltpu.make_async_copy(
            obuf.at[b], o_hbm.at[pl.ds(k * CH, CH), :], outsem.at[b])

    for b in range(NBUF):
        in_copy(b, b).start()

    for k in range(NCH):
        b = k % NBUF
        in_copy(k, b).wait()
        x = xbuf[b]
        logits = jax.lax.dot_general(
            x, w_ref[...],
            dimension_numbers=(((1,), (1,)), ((), ())),
            precision=jax.lax.Precision.DEFAULT,
            preferred_element_type=jnp.float32,
        ) + b_ref[...]
        u = u_ref[pl.ds(k * CH, CH), :]
        g = -jnp.log(-jnp.log(u + 1e-10) + 1e-10)
        z = (logits + g) / 0.5
        mask = (z[:, 1] > z[:, 0]).astype(x.dtype)
        if k >= NBUF:
            out_copy(k - NBUF, b).wait()
        obuf[b] = x * mask[:, None]
        out_copy(k, b).start()
        if k + NBUF < NCH:
            in_copy(k + NBUF, b).start()

    for k in range(NCH - NBUF, NCH):
        out_copy(k, k % NBUF).wait()


def kernel(x, W, b, gumbel_u):
    b2 = b.reshape(1, 2)
    return pl.pallas_call(
        _body,
        in_specs=[
            pl.BlockSpec(memory_space=pltpu.VMEM),
            pl.BlockSpec(memory_space=pltpu.VMEM),
            pl.BlockSpec(memory_space=pltpu.VMEM),
            pl.BlockSpec(memory_space=pl.ANY),
        ],
        out_specs=pl.BlockSpec(memory_space=pl.ANY),
        out_shape=jax.ShapeDtypeStruct((N, D), x.dtype),
        scratch_shapes=[
            pltpu.VMEM((NBUF, CH, D), jnp.float32),
            pltpu.VMEM((NBUF, CH, D), jnp.float32),
            pltpu.SemaphoreType.DMA((NBUF,)),
            pltpu.SemaphoreType.DMA((NBUF,)),
        ],
    )(W, b2, gumbel_u, x)


# speedup vs baseline: 1.0072x; 1.0072x over previous
import jax
import jax.numpy as jnp
from jax.experimental import pallas as pl
from jax.experimental.pallas import tpu as pltpu

N = 16384
D = 1024
CH = 1024
NBUF = 4
NCH = N // CH


def _body(w_ref, b_ref, u_ref, x_hbm, o_hbm, xbuf, obuf, insem, outsem):
    def in_copy(k, b):
        return pltpu.make_async_copy(
            x_hbm.at[pl.ds(k * CH, CH), :], xbuf.at[b], insem.at[b])

    def out_copy(k, b):
        return pltpu.make_async_copy(
            obuf.at[b], o_hbm.at[pl.ds(k * CH, CH), :], outsem.at[b])

    for b in range(NBUF):
        in_copy(b, b).start()

    for k in range(NCH):
        b = k % NBUF
        in_copy(k, b).wait()
        x = xbuf[b]
        logits = jax.lax.dot_general(
            x, w_ref[...],
            dimension_numbers=(((1,), (1,)), ((), ())),
            precision=jax.lax.Precision.DEFAULT,
            preferred_element_type=jnp.float32,
        ) + b_ref[...]
        u = u_ref[pl.ds(k * CH, CH), :]
        g = -jnp.log(-jnp.log(u + 1e-10) + 1e-10)
        z = (logits + g) / 0.5
        mask = (z[:, 1] > z[:, 0]).astype(x.dtype)
        if k >= NBUF:
            out_copy(k - NBUF, b).wait()
        obuf[b] = x * mask[:, None]
        out_copy(k, b).start()
        if k + NBUF < NCH:
            in_copy(k + NBUF, b).start()

    for k in range(NCH - NBUF, NCH):
        out_copy(k, k % NBUF).wait()


def kernel(x, W, b, gumbel_u):
    b2 = b.reshape(1, 2)
    return pl.pallas_call(
        _body,
        in_specs=[
            pl.BlockSpec(memory_space=pltpu.VMEM),
            pl.BlockSpec(memory_space=pltpu.VMEM),
            pl.BlockSpec(memory_space=pltpu.VMEM),
            pl.BlockSpec(memory_space=pl.ANY),
        ],
        out_specs=pl.BlockSpec(memory_space=pl.ANY),
        out_shape=jax.ShapeDtypeStruct((N, D), x.dtype),
        scratch_shapes=[
            pltpu.VMEM((NBUF, CH, D), jnp.float32),
            pltpu.VMEM((NBUF, CH, D), jnp.float32),
            pltpu.SemaphoreType.DMA((NBUF,)),
            pltpu.SemaphoreType.DMA((NBUF,)),
        ],
    )(W, b2, gumbel_u, x)
